# baseline (speedup 1.0000x reference)
import functools
from itertools import combinations

import jax
import jax.numpy as jnp
import numpy as np
from jax import lax
from jax.experimental import pallas as pl
from jax.experimental.pallas import tpu as pltpu
from jax.experimental.pallas import tpu_sc as plsc

N_RADIAL, N_ANGULAR = 5, 8
N_VERTICES, N_NEIGHBORS = 2048, 8
N_PTS = N_RADIAL * N_ANGULAR
TRI = np.array(list(combinations(range(N_NEIGHBORS), 3)), dtype=np.int32)
N_TRI = TRI.shape[0]

NC, NS, L = 2, 16, 16
NW = NC * NS
VPW = N_VERTICES // NW
NK = 4
NPV = 3
ROW = 2 * N_NEIGHBORS
OUTW = 3 * N_PTS

i32 = jnp.int32
f32 = np.float32
INF = np.float32(np.inf)
MAGIC = np.int32(0x5F3759DF)


def _bcast(v):
    return jnp.broadcast_to(v, (L,))


def _ibc(v):
    return jnp.broadcast_to(np.int32(v), (L,))


def _sqrt(x):
    b = lax.bitcast_convert_type(x, i32)
    y = lax.bitcast_convert_type(MAGIC - jnp.right_shift(b, 1), f32)
    xh = x * f32(0.5)
    y = y * (f32(1.5) - xh * y * y)
    y = y * (f32(1.5) - xh * y * y)
    return x * y


def _sc_body(pxy_hbm, tpl_hbm, itab_hbm, outw_hbm, outi_hbm,
             pxy_v, tpl_v, itab_v,
             a1x_v, a1y_v, a2x_v, a2y_v, ax_v, ay_v, pen_v, sn_v, ow_v, oi_v):
    wid = lax.axis_index("s") * NC + lax.axis_index("c")
    base = wid * VPW
    pltpu.sync_copy(pxy_hbm.at[pl.ds(base * ROW, VPW * ROW)], pxy_v)
    pltpu.sync_copy(tpl_hbm, tpl_v)
    pltpu.sync_copy(itab_hbm, itab_v)

    txs = tuple(tpl_v[pl.ds(pv * L, L)] for pv in range(NPV))
    tys = tuple(tpl_v[pl.ds((NPV + pv) * L, L)] for pv in range(NPV))
    lane = lax.broadcasted_iota(i32, (L,), 0)
    m8 = lane < N_ANGULAR
    col3 = lane * 3

    def vertex_body(j, _):
        jspl = _bcast(j)
        jbase = jspl * ROW
        jout = jspl * OUTW

        pxs = [plsc.load_gather(pxy_v, [jbase + _ibc(2 * p)])
               for p in range(N_NEIGHBORS)]
        pys = [plsc.load_gather(pxy_v, [jbase + _ibc(2 * p + 1)])
               for p in range(N_NEIGHBORS)]

        for k in range(NK):
            cax = itab_v[pl.ds(k * L, L)]
            cbx = itab_v[pl.ds((NK + k) * L, L)]
            ccx = itab_v[pl.ds((2 * NK + k) * L, L)]
            ax = plsc.load_gather(pxy_v, [jbase + cax])
            ay = plsc.load_gather(pxy_v, [jbase + cax + 1])
            bx = plsc.load_gather(pxy_v, [jbase + cbx])
            by = plsc.load_gather(pxy_v, [jbase + cbx + 1])
            cx = plsc.load_gather(pxy_v, [jbase + ccx])
            cy = plsc.load_gather(pxy_v, [jbase + ccx + 1])
            v0x = cx - ax; v0y = cy - ay
            v1x = bx - ax; v1y = by - ay
            d00 = v0x * v0x + v0y * v0y
            d01 = v0x * v1x + v0y * v1y
            d11 = v1x * v1x + v1y * v1y
            ivd = f32(1.0) / (d00 * d11 - d01 * d01)
            orient = v1x * v0y - v1y * v0x
            acc = None
            for p in range(N_NEIGHBORS):
                pxp = pxs[p]; pyp = pys[p]
                dxa = ax - pxp; dya = ay - pyp
                dxb = bx - pxp; dyb = by - pyp
                dxc = cx - pxp; dyc = cy - pyp
                za = dxa * dxa + dya * dya
                zb = dxb * dxb + dyb * dyb
                zc = dxc * dxc + dyc * dyc
                det = (dxa * (dyb * zc - dyc * zb)
                       - dya * (dxb * zc - dxc * zb)
                       + za * (dxb * dyc - dxc * dyb))
                c = orient * det > f32(0.0)
                acc = c if acc is None else (acc | c)
            pen = jnp.where(acc, INF, f32(0.0))
            a2x = (d11 * v0x - d01 * v1x) * ivd
            a2y = (d11 * v0y - d01 * v1y) * ivd
            a1x = (d00 * v1x - d01 * v0x) * ivd
            a1y = (d00 * v1y - d01 * v0y) * ivd
            sl = pl.ds(k * L, L)
            a1x_v[sl] = a1x; a1y_v[sl] = a1y
            a2x_v[sl] = a2x; a2y_v[sl] = a2y
            ax_v[sl] = ax; ay_v[sl] = ay
            pen_v[sl] = pen

        for p in range(N_NEIGHBORS):
            for pv in range(NPV):
                dx = pxs[p] - txs[pv]
                dy = pys[p] - tys[pv]
                sn_v[pl.ds((p * NPV + pv) * L, L)] = _sqrt(dx * dx + dy * dy)

        lanepv = tuple(lane + np.int32(pv * L) for pv in range(NPV))

        def tri_body(t, carry):
            tspl = carry[2 * NPV]
            a1x = plsc.load_gather(a1x_v, [tspl])
            a1y = plsc.load_gather(a1y_v, [tspl])
            a2x = plsc.load_gather(a2x_v, [tspl])
            a2y = plsc.load_gather(a2y_v, [tspl])
            axs = plsc.load_gather(ax_v, [tspl])
            ays = plsc.load_gather(ay_v, [tspl])
            pens = plsc.load_gather(pen_v, [tspl])
            b0 = jnp.right_shift(plsc.load_gather(itab_v, [tspl]),
                                 1) * (NPV * L)
            b1 = jnp.right_shift(
                plsc.load_gather(itab_v, [tspl + NK * L]), 1) * (NPV * L)
            b2 = jnp.right_shift(
                plsc.load_gather(itab_v, [tspl + 2 * NK * L]), 1) * (NPV * L)
            out = []
            for pv in range(NPV):
                bd, bt = carry[pv], carry[NPV + pv]
                v2x = txs[pv] - axs
                v2y = tys[pv] - ays
                w1 = a1x * v2x + a1y * v2y
                w2 = a2x * v2x + a2y * v2y
                w0 = f32(1.0) - w1 - w2
                wmin = jnp.minimum(jnp.minimum(w0, w1), w2)
                wmax = jnp.maximum(jnp.maximum(w0, w1), w2)
                inside = (wmin > f32(0.0)) & (wmax < f32(1.0))
                dist = (plsc.load_gather(sn_v, [b0 + lanepv[pv]])
                        + plsc.load_gather(sn_v, [b1 + lanepv[pv]])
                        + plsc.load_gather(sn_v, [b2 + lanepv[pv]]))
                de = jnp.where(inside, dist + pens, INF)
                upd = de < bd
                out.append((jnp.where(upd, de, bd), jnp.where(upd, tspl, bt)))
            return (out[0][0], out[1][0], out[2][0],
                    out[0][1], out[1][1], out[2][1], tspl + 1)

        init = (_bcast(INF), _bcast(INF), _bcast(INF),
                _ibc(0), _ibc(0), _ibc(0), _ibc(0))
        carry = lax.fori_loop(np.int32(0), np.int32(N_TRI), tri_body, init,
                              unroll=8)
        bd = [carry[0], carry[1], carry[2]]
        bt = [carry[3], carry[4], carry[5]]

        for pv in range(NPV):
            g0 = jnp.right_shift(plsc.load_gather(itab_v, [bt[pv]]), 1)
            g1 = jnp.right_shift(
                plsc.load_gather(itab_v, [bt[pv] + NK * L]), 1)
            g2 = jnp.right_shift(
                plsc.load_gather(itab_v, [bt[pv] + 2 * NK * L]), 1)
            a1x = plsc.load_gather(a1x_v, [bt[pv]])
            a1y = plsc.load_gather(a1y_v, [bt[pv]])
            a2x = plsc.load_gather(a2x_v, [bt[pv]])
            a2y = plsc.load_gather(a2y_v, [bt[pv]])
            axg = plsc.load_gather(ax_v, [bt[pv]])
            ayg = plsc.load_gather(ay_v, [bt[pv]])
            v2x = txs[pv] - axg
            v2y = tys[pv] - ayg
            w1 = a1x * v2x + a1y * v2y
            w2 = a2x * v2x + a2y * v2y
            w0 = f32(1.0) - w1 - w2
            neg = bd[pv] == INF
            w0 = jnp.where(neg, f32(0.0), w0)
            w1 = jnp.where(neg, f32(0.0), w1)
            w2 = jnp.where(neg, f32(0.0), w2)
            g0 = jnp.where(neg, 0, g0)
            g1 = jnp.where(neg, 0, g1)
            g2 = jnp.where(neg, 0, g2)
            cb = jout + col3 + np.int32(3 * L * pv)
            if pv < 2:
                plsc.store_scatter(ow_v, [cb], w0)
                plsc.store_scatter(ow_v, [cb + 1], w1)
                plsc.store_scatter(ow_v, [cb + 2], w2)
                plsc.store_scatter(oi_v, [cb], g0)
                plsc.store_scatter(oi_v, [cb + 1], g1)
                plsc.store_scatter(oi_v, [cb + 2], g2)
            else:
                plsc.store_scatter(ow_v, [cb], w0, mask=m8)
                plsc.store_scatter(ow_v, [cb + 1], w1, mask=m8)
                plsc.store_scatter(ow_v, [cb + 2], w2, mask=m8)
                plsc.store_scatter(oi_v, [cb], g0, mask=m8)
                plsc.store_scatter(oi_v, [cb + 1], g1, mask=m8)
                plsc.store_scatter(oi_v, [cb + 2], g2, mask=m8)
        return _

    lax.fori_loop(i32(0), i32(VPW), vertex_body, i32(0))
    pltpu.sync_copy(ow_v, outw_hbm.at[pl.ds(base * OUTW, VPW * OUTW)])
    pltpu.sync_copy(oi_v, outi_hbm.at[pl.ds(base * OUTW, VPW * OUTW)])


def _build_itab():
    itab = np.zeros((3 * NK * L,), dtype=np.int32)
    for t in range(N_TRI):
        itab[t] = 2 * TRI[t, 0]
        itab[NK * L + t] = 2 * TRI[t, 1]
        itab[2 * NK * L + t] = 2 * TRI[t, 2]
    return itab


_ITAB = _build_itab()


@jax.jit
def _run(template, projections):
    pxy = projections.reshape(N_VERTICES * ROW)
    t2 = template.reshape(N_PTS, 2)
    tpl = jnp.concatenate([
        jnp.pad(t2[:, 0], (0, NPV * L - N_PTS)),
        jnp.pad(t2[:, 1], (0, NPV * L - N_PTS)),
    ])
    itab = jnp.asarray(_ITAB)

    mesh = plsc.VectorSubcoreMesh(core_axis_name="c", subcore_axis_name="s")
    sc = functools.partial(
        pl.kernel,
        mesh=mesh,
        compiler_params=pltpu.CompilerParams(needs_layout_passes=False),
        out_type=[
            jax.ShapeDtypeStruct((N_VERTICES * OUTW,), jnp.float32),
            jax.ShapeDtypeStruct((N_VERTICES * OUTW,), jnp.int32),
        ],
        scratch_types=[
            pltpu.VMEM((VPW * ROW,), jnp.float32),
            pltpu.VMEM((2 * NPV * L,), jnp.float32),
            pltpu.VMEM((3 * NK * L,), jnp.int32),
            pltpu.VMEM((NK * L,), jnp.float32),
            pltpu.VMEM((NK * L,), jnp.float32),
            pltpu.VMEM((NK * L,), jnp.float32),
            pltpu.VMEM((NK * L,), jnp.float32),
            pltpu.VMEM((NK * L,), jnp.float32),
            pltpu.VMEM((NK * L,), jnp.float32),
            pltpu.VMEM((NK * L,), jnp.float32),
            pltpu.VMEM((N_NEIGHBORS * NPV * L,), jnp.float32),
            pltpu.VMEM((VPW * OUTW,), jnp.float32),
            pltpu.VMEM((VPW * OUTW,), jnp.int32),
        ],
    )(_sc_body)
    outw, outi = sc(pxy, tpl, itab)
    bc = outw.reshape(N_VERTICES, N_RADIAL, N_ANGULAR, 3).astype(jnp.float64)
    idx = outi.reshape(N_VERTICES, N_RADIAL, N_ANGULAR, 3)
    return bc, idx


def kernel(template, projections):
    return _run(template, projections)


# speedup vs baseline: 1.2407x; 1.2407x over previous
import functools
from itertools import combinations

import jax
import jax.numpy as jnp
import numpy as np
from jax import lax
from jax.experimental import pallas as pl
from jax.experimental.pallas import tpu as pltpu
from jax.experimental.pallas import tpu_sc as plsc

N_RADIAL, N_ANGULAR = 5, 8
N_VERTICES, N_NEIGHBORS = 2048, 8
N_PTS = N_RADIAL * N_ANGULAR
TRI = np.array(list(combinations(range(N_NEIGHBORS), 3)), dtype=np.int32)
N_TRI = TRI.shape[0]

NC, NS, L = 2, 16, 16
NW = NC * NS
VPW = N_VERTICES // NW
NK = 4
NPV = 3
ROW = 2 * N_NEIGHBORS
OUTW = 3 * N_PTS

i32 = jnp.int32
f32 = np.float32
INF = np.float32(np.inf)
MAGIC = np.int32(0x5F3759DF)


def _bcast(v):
    return jnp.broadcast_to(v, (L,))


def _ibc(v):
    return jnp.broadcast_to(np.int32(v), (L,))


def _sqrt(x):
    b = lax.bitcast_convert_type(x, i32)
    y = lax.bitcast_convert_type(MAGIC - jnp.right_shift(b, 1), f32)
    xh = x * f32(0.5)
    y = y * (f32(1.5) - xh * y * y)
    y = y * (f32(1.5) - xh * y * y)
    return x * y


def _sc_body(pxy_hbm, tpl_hbm, itab_hbm, outw_hbm, outi_hbm,
             pxy_v, tpl_v, itab_v,
             a1x_v, a1y_v, a2x_v, a2y_v, ax_v, ay_v, pen_v, sn_v, ow_v, oi_v):
    wid = lax.axis_index("s") * NC + lax.axis_index("c")
    base = wid * VPW
    pltpu.sync_copy(pxy_hbm.at[pl.ds(base * ROW, VPW * ROW)], pxy_v)
    pltpu.sync_copy(tpl_hbm, tpl_v)
    pltpu.sync_copy(itab_hbm, itab_v)

    txs = tuple(tpl_v[pl.ds(pv * L, L)] for pv in range(NPV))
    tys = tuple(tpl_v[pl.ds((NPV + pv) * L, L)] for pv in range(NPV))
    lane = lax.broadcasted_iota(i32, (L,), 0)
    m8 = lane < N_ANGULAR
    col3 = lane * 3

    def vertex_body(j, _):
        jspl = _bcast(j)
        jbase = jspl * ROW
        jout = jspl * OUTW

        pxs = [plsc.load_gather(pxy_v, [jbase + _ibc(2 * p)])
               for p in range(N_NEIGHBORS)]
        pys = [plsc.load_gather(pxy_v, [jbase + _ibc(2 * p + 1)])
               for p in range(N_NEIGHBORS)]

        for k in range(NK):
            cax = itab_v[pl.ds(k * L, L)]
            cbx = itab_v[pl.ds((NK + k) * L, L)]
            ccx = itab_v[pl.ds((2 * NK + k) * L, L)]
            ax = plsc.load_gather(pxy_v, [jbase + cax])
            ay = plsc.load_gather(pxy_v, [jbase + cax + 1])
            bx = plsc.load_gather(pxy_v, [jbase + cbx])
            by = plsc.load_gather(pxy_v, [jbase + cbx + 1])
            cx = plsc.load_gather(pxy_v, [jbase + ccx])
            cy = plsc.load_gather(pxy_v, [jbase + ccx + 1])
            v0x = cx - ax; v0y = cy - ay
            v1x = bx - ax; v1y = by - ay
            d00 = v0x * v0x + v0y * v0y
            d01 = v0x * v1x + v0y * v1y
            d11 = v1x * v1x + v1y * v1y
            ivd = f32(1.0) / (d00 * d11 - d01 * d01)
            orient = v1x * v0y - v1y * v0x
            acc = None
            for p in range(N_NEIGHBORS):
                pxp = pxs[p]; pyp = pys[p]
                dxa = ax - pxp; dya = ay - pyp
                dxb = bx - pxp; dyb = by - pyp
                dxc = cx - pxp; dyc = cy - pyp
                za = dxa * dxa + dya * dya
                zb = dxb * dxb + dyb * dyb
                zc = dxc * dxc + dyc * dyc
                det = (dxa * (dyb * zc - dyc * zb)
                       - dya * (dxb * zc - dxc * zb)
                       + za * (dxb * dyc - dxc * dyb))
                c = orient * det > f32(0.0)
                acc = c if acc is None else (acc | c)
            pen = jnp.where(acc, INF, f32(0.0))
            a2x = (d11 * v0x - d01 * v1x) * ivd
            a2y = (d11 * v0y - d01 * v1y) * ivd
            a1x = (d00 * v1x - d01 * v0x) * ivd
            a1y = (d00 * v1y - d01 * v0y) * ivd
            sl = pl.ds(k * L, L)
            a1x_v[sl] = a1x; a1y_v[sl] = a1y
            a2x_v[sl] = a2x; a2y_v[sl] = a2y
            ax_v[sl] = ax; ay_v[sl] = ay
            pen_v[sl] = pen

        for p in range(N_NEIGHBORS):
            for pv in range(NPV):
                dx = pxs[p] - txs[pv]
                dy = pys[p] - tys[pv]
                sn_v[pl.ds((p * NPV + pv) * L, L)] = _sqrt(dx * dx + dy * dy)

        lanepv = tuple(lane + np.int32(pv * L) for pv in range(NPV))

        def tri_body(t, carry):
            tspl = carry[2 * NPV]
            a1x = plsc.load_gather(a1x_v, [tspl])
            a1y = plsc.load_gather(a1y_v, [tspl])
            a2x = plsc.load_gather(a2x_v, [tspl])
            a2y = plsc.load_gather(a2y_v, [tspl])
            axs = plsc.load_gather(ax_v, [tspl])
            ays = plsc.load_gather(ay_v, [tspl])
            pens = plsc.load_gather(pen_v, [tspl])
            b0 = jnp.right_shift(plsc.load_gather(itab_v, [tspl]),
                                 1) * (NPV * L)
            b1 = jnp.right_shift(
                plsc.load_gather(itab_v, [tspl + NK * L]), 1) * (NPV * L)
            b2 = jnp.right_shift(
                plsc.load_gather(itab_v, [tspl + 2 * NK * L]), 1) * (NPV * L)
            out = []
            for pv in range(NPV):
                bd, bt = carry[pv], carry[NPV + pv]
                v2x = txs[pv] - axs
                v2y = tys[pv] - ays
                w1 = a1x * v2x + a1y * v2y
                w2 = a2x * v2x + a2y * v2y
                w0 = f32(1.0) - w1 - w2
                wmin = jnp.minimum(jnp.minimum(w0, w1), w2)
                wmax = jnp.maximum(jnp.maximum(w0, w1), w2)
                inside = (wmin > f32(0.0)) & (wmax < f32(1.0))
                dist = (plsc.load_gather(sn_v, [b0 + lanepv[pv]])
                        + plsc.load_gather(sn_v, [b1 + lanepv[pv]])
                        + plsc.load_gather(sn_v, [b2 + lanepv[pv]]))
                de = jnp.where(inside, dist + pens, INF)
                upd = de < bd
                out.append((jnp.where(upd, de, bd), jnp.where(upd, tspl, bt)))
            return (out[0][0], out[1][0], out[2][0],
                    out[0][1], out[1][1], out[2][1], tspl + 1)

        init = (_bcast(INF), _bcast(INF), _bcast(INF),
                _ibc(0), _ibc(0), _ibc(0), _ibc(0))
        carry = lax.fori_loop(np.int32(0), np.int32(N_TRI), tri_body, init,
                              unroll=2)
        bd = [carry[0], carry[1], carry[2]]
        bt = [carry[3], carry[4], carry[5]]

        for pv in range(NPV):
            g0 = jnp.right_shift(plsc.load_gather(itab_v, [bt[pv]]), 1)
            g1 = jnp.right_shift(
                plsc.load_gather(itab_v, [bt[pv] + NK * L]), 1)
            g2 = jnp.right_shift(
                plsc.load_gather(itab_v, [bt[pv] + 2 * NK * L]), 1)
            a1x = plsc.load_gather(a1x_v, [bt[pv]])
            a1y = plsc.load_gather(a1y_v, [bt[pv]])
            a2x = plsc.load_gather(a2x_v, [bt[pv]])
            a2y = plsc.load_gather(a2y_v, [bt[pv]])
            axg = plsc.load_gather(ax_v, [bt[pv]])
            ayg = plsc.load_gather(ay_v, [bt[pv]])
            v2x = txs[pv] - axg
            v2y = tys[pv] - ayg
            w1 = a1x * v2x + a1y * v2y
            w2 = a2x * v2x + a2y * v2y
            w0 = f32(1.0) - w1 - w2
            neg = bd[pv] == INF
            w0 = jnp.where(neg, f32(0.0), w0)
            w1 = jnp.where(neg, f32(0.0), w1)
            w2 = jnp.where(neg, f32(0.0), w2)
            g0 = jnp.where(neg, 0, g0)
            g1 = jnp.where(neg, 0, g1)
            g2 = jnp.where(neg, 0, g2)
            cb = jout + col3 + np.int32(3 * L * pv)
            if pv < 2:
                plsc.store_scatter(ow_v, [cb], w0)
                plsc.store_scatter(ow_v, [cb + 1], w1)
                plsc.store_scatter(ow_v, [cb + 2], w2)
                plsc.store_scatter(oi_v, [cb], g0)
                plsc.store_scatter(oi_v, [cb + 1], g1)
                plsc.store_scatter(oi_v, [cb + 2], g2)
            else:
                plsc.store_scatter(ow_v, [cb], w0, mask=m8)
                plsc.store_scatter(ow_v, [cb + 1], w1, mask=m8)
                plsc.store_scatter(ow_v, [cb + 2], w2, mask=m8)
                plsc.store_scatter(oi_v, [cb], g0, mask=m8)
                plsc.store_scatter(oi_v, [cb + 1], g1, mask=m8)
                plsc.store_scatter(oi_v, [cb + 2], g2, mask=m8)
        return _

    lax.fori_loop(i32(0), i32(VPW), vertex_body, i32(0))
    pltpu.sync_copy(ow_v, outw_hbm.at[pl.ds(base * OUTW, VPW * OUTW)])
    pltpu.sync_copy(oi_v, outi_hbm.at[pl.ds(base * OUTW, VPW * OUTW)])


def _build_itab():
    itab = np.zeros((3 * NK * L,), dtype=np.int32)
    for t in range(N_TRI):
        itab[t] = 2 * TRI[t, 0]
        itab[NK * L + t] = 2 * TRI[t, 1]
        itab[2 * NK * L + t] = 2 * TRI[t, 2]
    return itab


_ITAB = _build_itab()


@jax.jit
def _run(template, projections):
    pxy = projections.reshape(N_VERTICES * ROW)
    t2 = template.reshape(N_PTS, 2)
    tpl = jnp.concatenate([
        jnp.pad(t2[:, 0], (0, NPV * L - N_PTS)),
        jnp.pad(t2[:, 1], (0, NPV * L - N_PTS)),
    ])
    itab = jnp.asarray(_ITAB)

    mesh = plsc.VectorSubcoreMesh(core_axis_name="c", subcore_axis_name="s")
    sc = functools.partial(
        pl.kernel,
        mesh=mesh,
        compiler_params=pltpu.CompilerParams(needs_layout_passes=False),
        out_type=[
            jax.ShapeDtypeStruct((N_VERTICES * OUTW,), jnp.float32),
            jax.ShapeDtypeStruct((N_VERTICES * OUTW,), jnp.int32),
        ],
        scratch_types=[
            pltpu.VMEM((VPW * ROW,), jnp.float32),
            pltpu.VMEM((2 * NPV * L,), jnp.float32),
            pltpu.VMEM((3 * NK * L,), jnp.int32),
            pltpu.VMEM((NK * L,), jnp.float32),
            pltpu.VMEM((NK * L,), jnp.float32),
            pltpu.VMEM((NK * L,), jnp.float32),
            pltpu.VMEM((NK * L,), jnp.float32),
            pltpu.VMEM((NK * L,), jnp.float32),
            pltpu.VMEM((NK * L,), jnp.float32),
            pltpu.VMEM((NK * L,), jnp.float32),
            pltpu.VMEM((N_NEIGHBORS * NPV * L,), jnp.float32),
            pltpu.VMEM((VPW * OUTW,), jnp.float32),
            pltpu.VMEM((VPW * OUTW,), jnp.int32),
        ],
    )(_sc_body)
    outw, outi = sc(pxy, tpl, itab)
    bc = outw.reshape(N_VERTICES, N_RADIAL, N_ANGULAR, 3).astype(jnp.float64)
    idx = outi.reshape(N_VERTICES, N_RADIAL, N_ANGULAR, 3)
    return bc, idx


def kernel(template, projections):
    return _run(template, projections)


# speedup vs baseline: 3.6283x; 2.9243x over previous
import functools
from itertools import combinations

import jax
import jax.numpy as jnp
import numpy as np
from jax import lax
from jax.experimental import pallas as pl
from jax.experimental.pallas import tpu as pltpu
from jax.experimental.pallas import tpu_sc as plsc

N_RADIAL, N_ANGULAR = 5, 8
N_VERTICES, N_NEIGHBORS = 2048, 8
N_PTS = N_RADIAL * N_ANGULAR
TRI = np.array(list(combinations(range(N_NEIGHBORS), 3)), dtype=np.int32)
N_TRI = TRI.shape[0]

NC, NS, L = 2, 16, 16
NW = NC * NS
VPW = N_VERTICES // NW
NK = 4
NPV = 3
ROW = 2 * N_NEIGHBORS
OUTW = 3 * N_PTS

i32 = jnp.int32
f32 = np.float32
INF = np.float32(np.inf)
MAGIC = np.int32(0x5F3759DF)


def _bcast(v):
    return jnp.broadcast_to(v, (L,))


def _ibc(v):
    return jnp.broadcast_to(np.int32(v), (L,))


def _widen_f64(x):
    b = lax.bitcast_convert_type(x, jnp.uint32)
    s = b & np.uint32(0x80000000)
    e = (b >> 23) & np.uint32(0xFF)
    m = b & np.uint32(0x7FFFFF)
    nz = e != 0
    hi = s | jnp.where(nz, ((e + 896) << 20) | (m >> 3), 0)
    lo = jnp.where(nz, m << 29, 0)
    pair = jnp.stack([lo, hi], axis=-1)
    return lax.bitcast_convert_type(pair, jnp.float64)


def _sqrt(x):
    b = lax.bitcast_convert_type(x, i32)
    y = lax.bitcast_convert_type(MAGIC - jnp.right_shift(b, 1), f32)
    xh = x * f32(0.5)
    y = y * (f32(1.5) - xh * y * y)
    y = y * (f32(1.5) - xh * y * y)
    return x * y


def _sc_body(pxy_hbm, tpl_hbm, itab_hbm, outw_hbm, outi_hbm,
             pxy_v, tpl_v, itab_v,
             a1x_v, a1y_v, a2x_v, a2y_v, ax_v, ay_v, pen_v, sn_v, ow_v, oi_v):
    wid = lax.axis_index("s") * NC + lax.axis_index("c")
    base = wid * VPW
    pltpu.sync_copy(pxy_hbm.at[pl.ds(base * ROW, VPW * ROW)], pxy_v)
    pltpu.sync_copy(tpl_hbm, tpl_v)
    pltpu.sync_copy(itab_hbm, itab_v)

    txs = tuple(tpl_v[pl.ds(pv * L, L)] for pv in range(NPV))
    tys = tuple(tpl_v[pl.ds((NPV + pv) * L, L)] for pv in range(NPV))
    lane = lax.broadcasted_iota(i32, (L,), 0)
    m8 = lane < N_ANGULAR
    col3 = lane * 3

    def vertex_body(j, _):
        jspl = _bcast(j)
        jbase = jspl * ROW
        jout = jspl * OUTW

        pxs = [plsc.load_gather(pxy_v, [jbase + _ibc(2 * p)])
               for p in range(N_NEIGHBORS)]
        pys = [plsc.load_gather(pxy_v, [jbase + _ibc(2 * p + 1)])
               for p in range(N_NEIGHBORS)]

        for k in range(NK):
            cax = itab_v[pl.ds(k * L, L)]
            cbx = itab_v[pl.ds((NK + k) * L, L)]
            ccx = itab_v[pl.ds((2 * NK + k) * L, L)]
            ax = plsc.load_gather(pxy_v, [jbase + cax])
            ay = plsc.load_gather(pxy_v, [jbase + cax + 1])
            bx = plsc.load_gather(pxy_v, [jbase + cbx])
            by = plsc.load_gather(pxy_v, [jbase + cbx + 1])
            cx = plsc.load_gather(pxy_v, [jbase + ccx])
            cy = plsc.load_gather(pxy_v, [jbase + ccx + 1])
            v0x = cx - ax; v0y = cy - ay
            v1x = bx - ax; v1y = by - ay
            d00 = v0x * v0x + v0y * v0y
            d01 = v0x * v1x + v0y * v1y
            d11 = v1x * v1x + v1y * v1y
            ivd = f32(1.0) / (d00 * d11 - d01 * d01)
            orient = v1x * v0y - v1y * v0x
            acc = None
            for p in range(N_NEIGHBORS):
                pxp = pxs[p]; pyp = pys[p]
                dxa = ax - pxp; dya = ay - pyp
                dxb = bx - pxp; dyb = by - pyp
                dxc = cx - pxp; dyc = cy - pyp
                za = dxa * dxa + dya * dya
                zb = dxb * dxb + dyb * dyb
                zc = dxc * dxc + dyc * dyc
                det = (dxa * (dyb * zc - dyc * zb)
                       - dya * (dxb * zc - dxc * zb)
                       + za * (dxb * dyc - dxc * dyb))
                c = orient * det > f32(0.0)
                acc = c if acc is None else (acc | c)
            pen = jnp.where(acc, INF, f32(0.0))
            a2x = (d11 * v0x - d01 * v1x) * ivd
            a2y = (d11 * v0y - d01 * v1y) * ivd
            a1x = (d00 * v1x - d01 * v0x) * ivd
            a1y = (d00 * v1y - d01 * v0y) * ivd
            sl = pl.ds(k * L, L)
            a1x_v[sl] = a1x; a1y_v[sl] = a1y
            a2x_v[sl] = a2x; a2y_v[sl] = a2y
            ax_v[sl] = ax; ay_v[sl] = ay
            pen_v[sl] = pen

        for p in range(N_NEIGHBORS):
            for pv in range(NPV):
                dx = pxs[p] - txs[pv]
                dy = pys[p] - tys[pv]
                sn_v[pl.ds((p * NPV + pv) * L, L)] = _sqrt(dx * dx + dy * dy)

        lanepv = tuple(lane + np.int32(pv * L) for pv in range(NPV))

        def tri_body(t, carry):
            tspl = carry[2 * NPV]
            a1x = plsc.load_gather(a1x_v, [tspl])
            a1y = plsc.load_gather(a1y_v, [tspl])
            a2x = plsc.load_gather(a2x_v, [tspl])
            a2y = plsc.load_gather(a2y_v, [tspl])
            axs = plsc.load_gather(ax_v, [tspl])
            ays = plsc.load_gather(ay_v, [tspl])
            pens = plsc.load_gather(pen_v, [tspl])
            b0 = jnp.right_shift(plsc.load_gather(itab_v, [tspl]),
                                 1) * (NPV * L)
            b1 = jnp.right_shift(
                plsc.load_gather(itab_v, [tspl + NK * L]), 1) * (NPV * L)
            b2 = jnp.right_shift(
                plsc.load_gather(itab_v, [tspl + 2 * NK * L]), 1) * (NPV * L)
            out = []
            for pv in range(NPV):
                bd, bt = carry[pv], carry[NPV + pv]
                v2x = txs[pv] - axs
                v2y = tys[pv] - ays
                w1 = a1x * v2x + a1y * v2y
                w2 = a2x * v2x + a2y * v2y
                w0 = f32(1.0) - w1 - w2
                wmin = jnp.minimum(jnp.minimum(w0, w1), w2)
                wmax = jnp.maximum(jnp.maximum(w0, w1), w2)
                inside = (wmin > f32(0.0)) & (wmax < f32(1.0))
                dist = (plsc.load_gather(sn_v, [b0 + lanepv[pv]])
                        + plsc.load_gather(sn_v, [b1 + lanepv[pv]])
                        + plsc.load_gather(sn_v, [b2 + lanepv[pv]]))
                de = jnp.where(inside, dist + pens, INF)
                upd = de < bd
                out.append((jnp.where(upd, de, bd), jnp.where(upd, tspl, bt)))
            return (out[0][0], out[1][0], out[2][0],
                    out[0][1], out[1][1], out[2][1], tspl + 1)

        init = (_bcast(INF), _bcast(INF), _bcast(INF),
                _ibc(0), _ibc(0), _ibc(0), _ibc(0))
        carry = lax.fori_loop(np.int32(0), np.int32(N_TRI), tri_body, init,
                              unroll=2)
        bd = [carry[0], carry[1], carry[2]]
        bt = [carry[3], carry[4], carry[5]]

        for pv in range(NPV):
            g0 = jnp.right_shift(plsc.load_gather(itab_v, [bt[pv]]), 1)
            g1 = jnp.right_shift(
                plsc.load_gather(itab_v, [bt[pv] + NK * L]), 1)
            g2 = jnp.right_shift(
                plsc.load_gather(itab_v, [bt[pv] + 2 * NK * L]), 1)
            a1x = plsc.load_gather(a1x_v, [bt[pv]])
            a1y = plsc.load_gather(a1y_v, [bt[pv]])
            a2x = plsc.load_gather(a2x_v, [bt[pv]])
            a2y = plsc.load_gather(a2y_v, [bt[pv]])
            axg = plsc.load_gather(ax_v, [bt[pv]])
            ayg = plsc.load_gather(ay_v, [bt[pv]])
            v2x = txs[pv] - axg
            v2y = tys[pv] - ayg
            w1 = a1x * v2x + a1y * v2y
            w2 = a2x * v2x + a2y * v2y
            w0 = f32(1.0) - w1 - w2
            neg = bd[pv] == INF
            w0 = jnp.where(neg, f32(0.0), w0)
            w1 = jnp.where(neg, f32(0.0), w1)
            w2 = jnp.where(neg, f32(0.0), w2)
            g0 = jnp.where(neg, 0, g0)
            g1 = jnp.where(neg, 0, g1)
            g2 = jnp.where(neg, 0, g2)
            cb = jout + col3 + np.int32(3 * L * pv)
            if pv < 2:
                plsc.store_scatter(ow_v, [cb], w0)
                plsc.store_scatter(ow_v, [cb + 1], w1)
                plsc.store_scatter(ow_v, [cb + 2], w2)
                plsc.store_scatter(oi_v, [cb], g0)
                plsc.store_scatter(oi_v, [cb + 1], g1)
                plsc.store_scatter(oi_v, [cb + 2], g2)
            else:
                plsc.store_scatter(ow_v, [cb], w0, mask=m8)
                plsc.store_scatter(ow_v, [cb + 1], w1, mask=m8)
                plsc.store_scatter(ow_v, [cb + 2], w2, mask=m8)
                plsc.store_scatter(oi_v, [cb], g0, mask=m8)
                plsc.store_scatter(oi_v, [cb + 1], g1, mask=m8)
                plsc.store_scatter(oi_v, [cb + 2], g2, mask=m8)
        return _

    lax.fori_loop(i32(0), i32(VPW), vertex_body, i32(0))
    pltpu.sync_copy(ow_v, outw_hbm.at[pl.ds(base * OUTW, VPW * OUTW)])
    pltpu.sync_copy(oi_v, outi_hbm.at[pl.ds(base * OUTW, VPW * OUTW)])


def _build_itab():
    itab = np.zeros((3 * NK * L,), dtype=np.int32)
    for t in range(N_TRI):
        itab[t] = 2 * TRI[t, 0]
        itab[NK * L + t] = 2 * TRI[t, 1]
        itab[2 * NK * L + t] = 2 * TRI[t, 2]
    return itab


_ITAB = _build_itab()


@jax.jit
def _run(template, projections):
    pxy = projections.reshape(N_VERTICES * ROW)
    t2 = template.reshape(N_PTS, 2)
    tpl = jnp.concatenate([
        jnp.pad(t2[:, 0], (0, NPV * L - N_PTS)),
        jnp.pad(t2[:, 1], (0, NPV * L - N_PTS)),
    ])
    itab = jnp.asarray(_ITAB)

    mesh = plsc.VectorSubcoreMesh(core_axis_name="c", subcore_axis_name="s")
    sc = functools.partial(
        pl.kernel,
        mesh=mesh,
        compiler_params=pltpu.CompilerParams(needs_layout_passes=False),
        out_type=[
            jax.ShapeDtypeStruct((N_VERTICES * OUTW,), jnp.float32),
            jax.ShapeDtypeStruct((N_VERTICES * OUTW,), jnp.int32),
        ],
        scratch_types=[
            pltpu.VMEM((VPW * ROW,), jnp.float32),
            pltpu.VMEM((2 * NPV * L,), jnp.float32),
            pltpu.VMEM((3 * NK * L,), jnp.int32),
            pltpu.VMEM((NK * L,), jnp.float32),
            pltpu.VMEM((NK * L,), jnp.float32),
            pltpu.VMEM((NK * L,), jnp.float32),
            pltpu.VMEM((NK * L,), jnp.float32),
            pltpu.VMEM((NK * L,), jnp.float32),
            pltpu.VMEM((NK * L,), jnp.float32),
            pltpu.VMEM((NK * L,), jnp.float32),
            pltpu.VMEM((N_NEIGHBORS * NPV * L,), jnp.float32),
            pltpu.VMEM((VPW * OUTW,), jnp.float32),
            pltpu.VMEM((VPW * OUTW,), jnp.int32),
        ],
    )(_sc_body)
    outw, outi = sc(pxy, tpl, itab)
    bc = _widen_f64(outw.reshape(N_VERTICES, N_RADIAL, N_ANGULAR, 3))
    idx = outi.reshape(N_VERTICES, N_RADIAL, N_ANGULAR, 3)
    return bc, idx


def kernel(template, projections):
    return _run(template, projections)
